# baseline (speedup 1.0000x reference)
import jax
import jax.numpy as jnp
from jax import lax
from jax.experimental import pallas as pl
from jax.experimental.pallas import tpu as pltpu
from jax.experimental.pallas import tpu_sc as plsc

N = 10000
E = 320000
C = 128
LR = 0.1
ALPHA = 0.1
ITERS = 16

HALF = 5000
NP = 5120
N2 = 2 * NP
ZROW = 5000
CH = 128
K1 = 21504
K2 = 10752
ROWS_PER_TILE = NP // 16

C1 = 0.5 * LR * ALPHA
CST = 2.0 / (N * C)


def _mesh():
    return plsc.VectorSubcoreMesh(core_axis_name="c", subcore_axis_name="s")


def _scale_rows(wx, rows, out, ch):
    def ebody(e, carry):
        w16 = jnp.squeeze(wx[pl.ds(e, 1)], axis=0)
        for g in range(8):
            sl = pl.ds(g * 16, 16)
            r16 = jnp.squeeze(rows[pl.ds(e, 1), sl], axis=0)
            out[pl.ds(e, 1), sl] = (w16 * r16)[None]
        return carry

    lax.fori_loop(0, ch, ebody, 0)


def _pass_body(tbl, th, wts, u_init, u_out, thb, wv, rows, rowsx, accu, sem):
    c = lax.axis_index("c")
    s = lax.axis_index("s")
    row = c * 16 + s
    off = s * ROWS_PER_TILE
    goff = c * NP + off
    pltpu.sync_copy(u_init.at[pl.ds(goff, ROWS_PER_TILE)],
                    accu.at[pl.ds(off, ROWS_PER_TILE)])
    plsc.subcore_barrier()

    def chunk(j, carry):
        base = j * CH
        pltpu.sync_copy(th.at[row, :, pl.ds(base, CH)], thb)
        pltpu.sync_copy(wts.at[row, pl.ds(base, CH)], wv)
        pltpu.async_copy(tbl.at[thb.at[0]], rows, sem).wait()
        _scale_rows(wv, rows, rowsx, CH)
        pltpu.sync_copy(rowsx, accu.at[thb.at[1]], add=True)
        return carry

    lax.fori_loop(0, K1 // CH, chunk, 0)
    plsc.subcore_barrier()
    pltpu.sync_copy(accu.at[pl.ds(off, ROWS_PER_TILE)],
                    u_out.at[pl.ds(goff, ROWS_PER_TILE)])


def _make_pass():
    return pl.kernel(
        _pass_body,
        out_type=jax.ShapeDtypeStruct((N2, C), jnp.float32),
        mesh=_mesh(),
        scratch_types=[
            pltpu.VMEM((2, CH), jnp.int32),
            pltpu.VMEM((CH, 16), jnp.float32),
            pltpu.VMEM((CH, C), jnp.float32),
            pltpu.VMEM((CH, C), jnp.float32),
            pltpu.VMEM_SHARED((NP, C), jnp.float32),
            pltpu.SemaphoreType.DMA,
        ],
    )


def _tc_matvec_body(p_ref, f_ref, w_ref, y_ref):
    t = jnp.dot(p_ref[...], w_ref[...],
                preferred_element_type=jnp.float32) - f_ref[...]
    y_ref[...] = lax.dot_general(
        t, w_ref[...], (((1,), (1,)), ((), ())),
        preferred_element_type=jnp.float32) * CST


def _tc_matvec(p, fpad, w):
    blk = 512
    return pl.pallas_call(
        _tc_matvec_body,
        grid=(N2 // blk,),
        in_specs=[
            pl.BlockSpec((blk, C), lambda i: (i, 0)),
            pl.BlockSpec((blk, C), lambda i: (i, 0)),
            pl.BlockSpec((C, C), lambda i: (0, 0)),
        ],
        out_specs=pl.BlockSpec((blk, C), lambda i: (i, 0)),
        out_shape=jax.ShapeDtypeStruct((N2, C), jnp.float32),
    )(p, fpad, w)


def _tc_update_body(x_ref, u_ref, d_ref, o_ref):
    scale = 1.0 - (LR * ALPHA) * (d_ref[:, :1] + 0.1)
    o_ref[...] = scale * x_ref[...] + u_ref[...]


def _tc_update(xp, u, d2):
    blk = 512
    return pl.pallas_call(
        _tc_update_body,
        grid=(N2 // blk,),
        in_specs=[
            pl.BlockSpec((blk, C), lambda i: (i, 0)),
            pl.BlockSpec((blk, C), lambda i: (i, 0)),
            pl.BlockSpec((blk, C), lambda i: (i, 0)),
        ],
        out_specs=pl.BlockSpec((blk, C), lambda i: (i, 0)),
        out_shape=jax.ShapeDtypeStruct((N2, C), jnp.float32),
    )(xp, u, d2)


def _layout(heads, tails, wts, k_per_tile):
    m = heads.shape[0]
    bsz = 16 * k_per_tile
    sc1 = heads >= NP
    order = jnp.argsort(sc1)
    hs = heads[order]
    ts = tails[order]
    ws = wts[order]
    n0 = m - jnp.sum(sc1.astype(jnp.int32))

    def block(lo_static, base_node, valid_fn):
        pos = lo_static + jnp.arange(bsz)
        h = lax.slice(hs, (lo_static,), (lo_static + bsz,))
        t = lax.slice(ts, (lo_static,), (lo_static + bsz,))
        w_ = lax.slice(ws, (lo_static,), (lo_static + bsz,))
        valid = valid_fn(pos)
        lh = jnp.clip(h - base_node, 0, NP - 1)
        t = jnp.where(valid, t, ZROW)
        w_ = jnp.where(valid, w_, 0.0)
        v = valid.astype(jnp.float32)
        resh = lambda x: x.reshape(k_per_tile, 16).T
        return resh(t), resh(lh), resh(w_), resh(v)

    t0, h0, w0, v0 = block(0, 0, lambda p: p < n0)
    t1, h1, w1, v1 = block(m - bsz, NP, lambda p: p >= n0)
    cat = lambda a, b: jnp.concatenate([a, b], axis=0)
    return cat(t0, t1), cat(h0, h1), cat(w0, w1), cat(v0, v1)


def _expand(a):
    return jnp.broadcast_to(a[:, :, None], (*a.shape, 16))


def kernel(forward_data, y, edge_index, edge_weight, W):
    del y
    src = edge_index[0]
    dst = edge_index[1]
    w = edge_weight

    def padid(x):
        return x + 120 * (x >= HALF).astype(jnp.int32)

    srcp = padid(src)
    dstp = padid(dst)

    heads1 = jnp.concatenate([dstp, srcp])
    tails1 = jnp.concatenate([srcp, dstp])
    a1 = jnp.concatenate([w, jnp.zeros((E,), jnp.float32)])
    T1, H1, A1, V1 = _layout(heads1, tails1, a1, K1)
    TH1 = jnp.stack([T1, H1], axis=1)
    A1x = _expand(A1)
    U1x = _expand(C1 * V1)

    padn = 16 * K1 - E
    heads2 = jnp.concatenate([srcp, jnp.full((padn,), NP + HALF, jnp.int32)])
    tails2 = jnp.concatenate([dstp, jnp.full((padn,), ZROW, jnp.int32)])
    w2 = jnp.concatenate([-LR * w, jnp.zeros((padn,), jnp.float32)])
    T2, H2, B2, V2p = _layout(heads2, tails2, w2, K1)
    TH2 = jnp.stack([T2, H2], axis=1)
    B2x = _expand(B2)

    xp = jnp.zeros((N2, C), jnp.float32)
    zz = jnp.zeros((N2, C), jnp.float32)
    oo = jnp.ones((N2, C), jnp.float32)
    fpad = jnp.zeros((N2, C), jnp.float32)
    fpad = lax.dynamic_update_slice(fpad, forward_data[:HALF], (0, 0))
    fpad = lax.dynamic_update_slice(fpad, forward_data[HALF:], (NP, 0))

    f = _make_pass()
    d2 = f(oo, TH2, _expand(V2p), zz)
    for _ in range(ITERS):
        p = f(xp, TH1, A1x, zz)
        u1 = f(xp, TH1, U1x, zz)
        yv = _tc_matvec(p, fpad, W)
        u = f(yv, TH2, B2x, u1)
        xp = _tc_update(xp, u, d2)

    return jnp.concatenate([xp[:HALF], xp[NP:NP + HALF]], axis=0)
